# Initial kernel scaffold; baseline (speedup 1.0000x reference)
import jax
import jax.numpy as jnp
from jax.experimental import pallas as pl


def kernel(hidden_states, router_w, w_gate_proj, w_up_proj, w_down_proj):
    raise NotImplementedError("write your pallas kernel here")


# speedup vs baseline: 1.8010x; 1.8010x over previous
import jax
import jax.numpy as jnp
from jax.experimental import pallas as pl
from functools import partial

_TN = 512


def _mlp_kernel(x_ref, wg_ref, wu_ref, wd_ref, o_ref):
    x = x_ref[...]
    g = jnp.dot(x, wg_ref[...], preferred_element_type=jnp.float32)
    u = jnp.dot(x, wu_ref[...], preferred_element_type=jnp.float32)
    a = (g * jax.nn.sigmoid(g)) * u
    o_ref[...] = jnp.dot(a, wd_ref[...], preferred_element_type=jnp.float32)


@jax.jit
def kernel(hidden_states, router_w, w_gate_proj, w_up_proj, w_down_proj):
    b, s, d = hidden_states.shape
    n = b * s
    ff = w_gate_proj.shape[1]
    x = hidden_states.reshape(n, d)
    out = pl.pallas_call(
        _mlp_kernel,
        grid=(n // _TN,),
        in_specs=[
            pl.BlockSpec((_TN, d), lambda i: (i, 0)),
            pl.BlockSpec((d, ff), lambda i: (0, 0)),
            pl.BlockSpec((d, ff), lambda i: (0, 0)),
            pl.BlockSpec((ff, d), lambda i: (0, 0)),
        ],
        out_specs=pl.BlockSpec((_TN, d), lambda i: (i, 0)),
        out_shape=jax.ShapeDtypeStruct((n, d), jnp.float32),
    )(x, w_gate_proj, w_up_proj, w_down_proj)
    return out.reshape(b, s, d)
